# baseline (speedup 1.0000x reference)
import functools

import jax
import jax.numpy as jnp
from jax import lax
from jax.experimental import pallas as pl
from jax.experimental.pallas import tpu as pltpu
from jax.experimental.pallas import tpu_sc as plsc

_NUM_FIELDS = 26
_VOCAB = 10000
_SUM_DIMS = _NUM_FIELDS * _VOCAB
_D = 16
_BATCH = 4096
_N_IDX = _BATCH * _NUM_FIELDS
_WINDOW = 128


def _gather_kernel(table_hbm, idx_hbm, out_hbm):
    def body(i_vmem, o_vmem):
        pltpu.sync_copy(table_hbm.at[i_vmem.at[0]], o_vmem)

    pltpu.emit_pipeline(
        body,
        grid=(_N_IDX // _WINDOW,),
        in_specs=[pl.BlockSpec((1, _WINDOW), lambda i: (0, i))],
        out_specs=[pl.BlockSpec((_WINDOW, _D), lambda i: (i, 0))],
        core_axis_name=("core", "subcore"),
        dimension_semantics=(pltpu.PARALLEL,),
    )(idx_hbm, out_hbm)


@jax.jit
def _sc_gather(table_flat, idx):
    mesh = plsc.VectorSubcoreMesh(core_axis_name="core", subcore_axis_name="subcore")
    k = pl.kernel(
        _gather_kernel,
        out_type=jax.ShapeDtypeStruct((_N_IDX, _D), jnp.float32),
        mesh=mesh,
        compiler_params=pltpu.CompilerParams(use_tc_tiling_on_sc=False),
    )
    return k(table_flat, idx)


def _tc_body(H_ref, M_ref, d_ref, W1_ref, b1_ref, g1_ref, be1_ref,
             W2_ref, b2_ref, g2_ref, be2_ref, W3_ref, b3_ref, out_ref):
    H = H_ref[...]
    G = jnp.dot(H, M_ref[...], preferred_element_type=jnp.float32)
    quad = jnp.sum(G * H, axis=1, keepdims=True)
    diag = jnp.sum(H * H * d_ref[...], axis=1, keepdims=True)
    fwfm = 0.5 * (quad - diag)
    h = jnp.dot(H, W1_ref[...], preferred_element_type=jnp.float32) + b1_ref[...]
    m = jnp.mean(h, axis=0, keepdims=True)
    v = jnp.mean((h - m) * (h - m), axis=0, keepdims=True)
    h = jnp.maximum(g1_ref[...] * (h - m) * lax.rsqrt(v + 1e-5) + be1_ref[...], 0.0)
    h = jnp.dot(h, W2_ref[...], preferred_element_type=jnp.float32) + b2_ref[...]
    m = jnp.mean(h, axis=0, keepdims=True)
    v = jnp.mean((h - m) * (h - m), axis=0, keepdims=True)
    h = jnp.maximum(g2_ref[...] * (h - m) * lax.rsqrt(v + 1e-5) + be2_ref[...], 0.0)
    o = jnp.dot(h, W3_ref[...], preferred_element_type=jnp.float32) + b3_ref[...]
    out_ref[...] = jax.nn.sigmoid(fwfm + o)


def kernel(x, emb_tables, field_cov_w, W1, b1, gamma1, beta1,
           W2, b2, gamma2, beta2, W3, b3):
    field_stride = _SUM_DIMS + _VOCAB
    idx = (x + field_stride * jnp.arange(_NUM_FIELDS, dtype=x.dtype)[None, :])
    idx = idx.reshape(1, _N_IDX)
    table_flat = emb_tables.reshape(_NUM_FIELDS * _SUM_DIMS, _D)

    rows = _sc_gather(table_flat, idx)
    H = rows.reshape(_BATCH, _NUM_FIELDS * _D)

    sym = (field_cov_w.T + field_cov_w) * 0.5
    M = jnp.kron(sym, jnp.eye(_D, dtype=jnp.float32))
    d = jnp.repeat(jnp.diagonal(sym), _D).reshape(1, -1)

    out = pl.pallas_call(
        _tc_body,
        out_shape=jax.ShapeDtypeStruct((_BATCH, 1), jnp.float32),
    )(H, M, d,
      W1, b1.reshape(1, -1), gamma1.reshape(1, -1), beta1.reshape(1, -1),
      W2, b2.reshape(1, -1), gamma2.reshape(1, -1), beta2.reshape(1, -1),
      W3, b3.reshape(1, -1))
    return out.reshape(_BATCH)


# speedup vs baseline: 8.8923x; 8.8923x over previous
import functools

import jax
import jax.numpy as jnp
from jax import lax
from jax.experimental import pallas as pl
from jax.experimental.pallas import tpu as pltpu
from jax.experimental.pallas import tpu_sc as plsc

_NUM_FIELDS = 26
_VOCAB = 10000
_SUM_DIMS = _NUM_FIELDS * _VOCAB
_D = 16
_BATCH = 4096
_N_IDX = _BATCH * _NUM_FIELDS
_WINDOW = 128


def _gather_kernel(table_hbm, idx_hbm, out_hbm):
    def body(i_vmem, o_vmem):
        pltpu.sync_copy(table_hbm.at[i_vmem.at[0]], o_vmem)

    pltpu.emit_pipeline(
        body,
        grid=(_N_IDX // _WINDOW,),
        in_specs=[pl.BlockSpec((1, _WINDOW), lambda i: (0, i))],
        out_specs=[pl.BlockSpec((_WINDOW, _D), lambda i: (i, 0))],
        core_axis_name=("core", "subcore"),
        dimension_semantics=(pltpu.PARALLEL,),
    )(idx_hbm, out_hbm)


@jax.jit
def _sc_gather(table_flat, idx):
    mesh = plsc.VectorSubcoreMesh(core_axis_name="core", subcore_axis_name="subcore")
    k = pl.kernel(
        _gather_kernel,
        out_type=jax.ShapeDtypeStruct((_N_IDX, _D), jnp.float32),
        mesh=mesh,
        compiler_params=pltpu.CompilerParams(use_tc_tiling_on_sc=False),
    )
    return k(table_flat, idx)


def _tc_body(H_ref, M_ref, d_ref, W1_ref, b1_ref, g1_ref, be1_ref,
             W2_ref, b2_ref, g2_ref, be2_ref, W3_ref, b3_ref, out_ref):
    H = H_ref[...]
    G = jnp.dot(H, M_ref[...], preferred_element_type=jnp.float32)
    quad = jnp.sum(G * H, axis=1, keepdims=True)
    diag = jnp.sum(H * H * d_ref[...], axis=1, keepdims=True)
    fwfm = 0.5 * (quad - diag)
    h = jnp.dot(H, W1_ref[...], preferred_element_type=jnp.float32) + b1_ref[...]
    m = jnp.mean(h, axis=0, keepdims=True)
    v = jnp.mean((h - m) * (h - m), axis=0, keepdims=True)
    h = jnp.maximum(g1_ref[...] * (h - m) * lax.rsqrt(v + 1e-5) + be1_ref[...], 0.0)
    h = jnp.dot(h, W2_ref[...], preferred_element_type=jnp.float32) + b2_ref[...]
    m = jnp.mean(h, axis=0, keepdims=True)
    v = jnp.mean((h - m) * (h - m), axis=0, keepdims=True)
    h = jnp.maximum(g2_ref[...] * (h - m) * lax.rsqrt(v + 1e-5) + be2_ref[...], 0.0)
    o = jnp.dot(h, W3_ref[...], preferred_element_type=jnp.float32) + b3_ref[...]
    out_ref[...] = jax.nn.sigmoid(fwfm + o)


def kernel(x, emb_tables, field_cov_w, W1, b1, gamma1, beta1,
           W2, b2, gamma2, beta2, W3, b3):
    compact = jnp.concatenate(
        [lax.slice(emb_tables, (i, _VOCAB * i, 0), (i + 1, _VOCAB * (i + 1), _D))
         for i in range(_NUM_FIELDS)], axis=1)[0]
    idx = (x + _VOCAB * jnp.arange(_NUM_FIELDS, dtype=x.dtype)[None, :])
    idx = idx.reshape(1, _N_IDX)

    rows = _sc_gather(compact, idx)
    H = rows.reshape(_BATCH, _NUM_FIELDS * _D)

    sym = (field_cov_w.T + field_cov_w) * 0.5
    M = jnp.kron(sym, jnp.eye(_D, dtype=jnp.float32))
    d = jnp.repeat(jnp.diagonal(sym), _D).reshape(1, -1)

    out = pl.pallas_call(
        _tc_body,
        out_shape=jax.ShapeDtypeStruct((_BATCH, 1), jnp.float32),
    )(H, M, d,
      W1, b1.reshape(1, -1), gamma1.reshape(1, -1), beta1.reshape(1, -1),
      W2, b2.reshape(1, -1), gamma2.reshape(1, -1), beta2.reshape(1, -1),
      W3, b3.reshape(1, -1))
    return out.reshape(_BATCH)
